# baseline (speedup 1.0000x reference)
import functools

import jax
import jax.numpy as jnp
from jax import lax
from jax.experimental import pallas as pl
from jax.experimental.pallas import tpu as pltpu
from jax.experimental.pallas import tpu_sc as plsc

HIDDEN = 128
BATCH = 4
SEQ = 2048

NC, NS, L = 2, 16, 16
NW = NC * NS
N = BATCH * SEQ
PPW = SEQ // NW
RPW = BATCH * PPW


@functools.partial(
    pl.kernel,
    out_type=jax.ShapeDtypeStruct((N, HIDDEN), jnp.float32),
    mesh=plsc.VectorSubcoreMesh(core_axis_name="c", subcore_axis_name="s"),
    scratch_types=[
        pltpu.VMEM((RPW,), jnp.int32),
        pltpu.VMEM((RPW, HIDDEN), jnp.float32),
        pltpu.VMEM_SHARED((NS, PPW, HIDDEN), jnp.float32),
        pltpu.SemaphoreType.DMA,
        [pltpu.SemaphoreType.DMA] * (2 * BATCH),
        [pltpu.SemaphoreType.DMA] * (2 * BATCH),
        pltpu.SemaphoreType.DMA,
    ],
)
def _embed_sum(ids_hbm, wtab_hbm, ptab_hbm, out_hbm, idx_v, rows_v, pos_sh,
               sem_i, sem_r, sem_g, sem_out):
    sid = lax.axis_index("s")
    wid = sid * NC + lax.axis_index("c")
    pbase = wid * PPW

    idx_copies = []
    for b in range(BATCH):
        idx_copies.append(
            pltpu.async_copy(
                ids_hbm.at[pl.ds(b * SEQ + pbase, PPW)],
                idx_v.at[pl.ds(b * PPW, PPW)],
                sem_i,
            )
        )

    pltpu.sync_copy(ptab_hbm.at[pl.ds(pbase, PPW)], pos_sh.at[sid])
    H = PPW // 2
    reps = []
    for k in range(2 * BATCH):
        b, h = divmod(k, 2)
        reps.append(
            pltpu.async_copy(
                pos_sh.at[sid, pl.ds(h * H, H), :],
                rows_v.at[pl.ds(b * PPW + h * H, H), :],
                sem_r[k],
            )
        )

    for c in idx_copies:
        c.wait()

    gathers = []
    for k in range(2 * BATCH):
        b, h = divmod(k, 2)
        sl = pl.ds(b * PPW + h * H, H)
        reps[k].wait()
        gathers.append(
            pltpu.async_copy(
                wtab_hbm.at[idx_v.at[sl]],
                rows_v.at[sl, :],
                sem_g[k],
                add=True,
            )
        )

    outs = []
    for k in range(2 * BATCH):
        b, h = divmod(k, 2)
        sl = pl.ds(b * PPW + h * H, H)
        gathers[k].wait()
        outs.append(
            pltpu.async_copy(
                rows_v.at[sl, :],
                out_hbm.at[pl.ds(b * SEQ + pbase + h * H, H)],
                sem_out,
            )
        )
    for o in outs:
        o.wait()


def kernel(input_ids, word_embeddings, position_embeddings):
    ids = input_ids.astype(jnp.int32).reshape(-1)
    out = _embed_sum(ids, word_embeddings, position_embeddings)
    return out.reshape(BATCH, SEQ, HIDDEN)


# speedup vs baseline: 1.0200x; 1.0200x over previous
import functools

import jax
import jax.numpy as jnp
from jax import lax
from jax.experimental import pallas as pl
from jax.experimental.pallas import tpu as pltpu
from jax.experimental.pallas import tpu_sc as plsc

HIDDEN = 128
BATCH = 4
SEQ = 2048

NC, NS, L = 2, 16, 16
NW = NC * NS
N = BATCH * SEQ
PPW = SEQ // NW
RPW = BATCH * PPW


@functools.partial(
    pl.kernel,
    out_type=jax.ShapeDtypeStruct((N, HIDDEN), jnp.float32),
    mesh=plsc.VectorSubcoreMesh(core_axis_name="c", subcore_axis_name="s"),
    scratch_types=[
        pltpu.VMEM((RPW,), jnp.int32),
        pltpu.VMEM((RPW, HIDDEN), jnp.float32),
        pltpu.VMEM_SHARED((NS, PPW, HIDDEN), jnp.float32),
        pltpu.SemaphoreType.DMA,
        [pltpu.SemaphoreType.DMA] * BATCH,
        [pltpu.SemaphoreType.DMA] * BATCH,
        pltpu.SemaphoreType.DMA,
    ],
)
def _embed_sum(ids_hbm, wtab_hbm, ptab_hbm, out_hbm, idx_v, rows_v, pos_sh,
               sem_i, sem_r, sem_g, sem_out):
    sid = lax.axis_index("s")
    wid = sid * NC + lax.axis_index("c")
    pbase = wid * PPW

    idx_copies = []
    for b in range(BATCH):
        idx_copies.append(
            pltpu.async_copy(
                ids_hbm.at[pl.ds(b * SEQ + pbase, PPW)],
                idx_v.at[pl.ds(b * PPW, PPW)],
                sem_i,
            )
        )

    pltpu.sync_copy(ptab_hbm.at[pl.ds(pbase, PPW)], pos_sh.at[sid])
    reps = []
    for b in range(BATCH):
        reps.append(
            pltpu.async_copy(
                pos_sh.at[sid],
                rows_v.at[pl.ds(b * PPW, PPW), :],
                sem_r[b],
            )
        )

    for c in idx_copies:
        c.wait()

    gathers = []
    for b in range(BATCH):
        sl = pl.ds(b * PPW, PPW)
        reps[b].wait()
        gathers.append(
            pltpu.async_copy(
                wtab_hbm.at[idx_v.at[sl]],
                rows_v.at[sl, :],
                sem_g[b],
                add=True,
            )
        )

    outs = []
    for b in range(BATCH):
        sl = pl.ds(b * PPW, PPW)
        gathers[b].wait()
        outs.append(
            pltpu.async_copy(
                rows_v.at[sl, :],
                out_hbm.at[pl.ds(b * SEQ + pbase, PPW)],
                sem_out,
            )
        )
    for o in outs:
        o.wait()


def kernel(input_ids, word_embeddings, position_embeddings):
    ids = input_ids.astype(jnp.int32).reshape(-1)
    out = _embed_sum(ids, word_embeddings, position_embeddings)
    return out.reshape(BATCH, SEQ, HIDDEN)
